# baseline (speedup 1.0000x reference)
import jax
import jax.numpy as jnp
from jax.experimental import pallas as pl
from jax.experimental.pallas import tpu as pltpu

_N_DMA = 8


def _copy_body(x_ref, o_ref, sems):
    n = x_ref.shape[1]
    chunk = n // _N_DMA
    for i in range(_N_DMA):
        pltpu.make_async_copy(
            x_ref.at[:, pl.ds(i * chunk, chunk)],
            o_ref.at[:, pl.ds(i * chunk, chunk)],
            sems.at[i],
        ).start()
    for i in range(_N_DMA):
        pltpu.make_async_copy(
            x_ref.at[:, pl.ds(i * chunk, chunk)],
            o_ref.at[:, pl.ds(i * chunk, chunk)],
            sems.at[i],
        ).wait()


def kernel(edge_index):
    return pl.pallas_call(
        _copy_body,
        in_specs=[pl.BlockSpec(memory_space=pltpu.MemorySpace.HBM)],
        out_specs=pl.BlockSpec(memory_space=pltpu.MemorySpace.HBM),
        scratch_shapes=[pltpu.SemaphoreType.DMA((_N_DMA,))],
        out_shape=jax.ShapeDtypeStruct(edge_index.shape, edge_index.dtype),
    )(edge_index)


# speedup vs baseline: 28.0937x; 28.0937x over previous
import functools

import jax
import jax.numpy as jnp
from jax import lax
from jax.experimental import pallas as pl
from jax.experimental.pallas import tpu as pltpu
from jax.experimental.pallas import tpu_sc as plsc

_LANE = 128
_CHUNK_TILES = 200


def kernel(edge_index):
    two, n_cols = edge_index.shape
    info = plsc.get_sparse_core_info()
    nc, ns = info.num_cores, info.num_subcores
    nw = nc * ns
    total_tiles = -(-n_cols // _LANE)
    tiles_pw = -(-total_tiles // nw)
    n_chunks = -(-tiles_pw // _CHUNK_TILES)
    mesh = plsc.VectorSubcoreMesh(core_axis_name="c", subcore_axis_name="s")

    @functools.partial(
        pl.kernel,
        mesh=mesh,
        out_type=jax.ShapeDtypeStruct((two, n_cols), edge_index.dtype),
        scratch_types=[
            pltpu.VMEM((two, _CHUNK_TILES * _LANE), jnp.int32),
            pltpu.VMEM((two, _CHUNK_TILES * _LANE), jnp.int32),
            pltpu.SemaphoreType.DMA((2,)),
            pltpu.SemaphoreType.DMA((2,)),
        ],
    )
    def _copy(x_hbm, o_hbm, b0, b1, sin, sout):
        wid = lax.axis_index("s") * nc + lax.axis_index("c")
        start_tile = jnp.minimum(wid * tiles_pw, total_tiles - tiles_pw)
        base = pl.multiple_of(start_tile * _LANE, _LANE)
        bufs = (b0, b1)

        def chunk_cols(j):
            return min(_CHUNK_TILES, tiles_pw - j * _CHUNK_TILES) * _LANE

        def in_dma(j):
            sz = chunk_cols(j)
            return pltpu.make_async_copy(
                x_hbm.at[:, pl.ds(base + j * _CHUNK_TILES * _LANE, sz)],
                bufs[j % 2].at[:, pl.ds(0, sz)],
                sin.at[j % 2],
            )

        def out_dma(j):
            sz = chunk_cols(j)
            return pltpu.make_async_copy(
                bufs[j % 2].at[:, pl.ds(0, sz)],
                o_hbm.at[:, pl.ds(base + j * _CHUNK_TILES * _LANE, sz)],
                sout.at[j % 2],
            )

        in_dma(0).start()
        for j in range(n_chunks):
            in_dma(j).wait()
            if j + 1 < n_chunks:
                if j >= 1:
                    out_dma(j - 1).wait()
                in_dma(j + 1).start()
            out_dma(j).start()
        out_dma(n_chunks - 1).wait()
        if n_chunks >= 2:
            out_dma(n_chunks - 2).wait()

    return _copy(edge_index)


# speedup vs baseline: 43.9720x; 1.5652x over previous
import jax
import jax.numpy as jnp
from jax.experimental import pallas as pl
from jax.experimental.pallas import tpu as pltpu

_CHUNK_COLS = 256000
_NBUF = 6
_LEAD = 2


def _make_body(n_chunks, chunk_cols, nbuf, lead):
    def body(x_ref, o_ref, *rest):
        bufs = rest[:nbuf]
        sin, sout = rest[nbuf], rest[nbuf + 1]

        def in_dma(j):
            return pltpu.make_async_copy(
                x_ref.at[:, pl.ds(j * chunk_cols, chunk_cols)],
                bufs[j % nbuf],
                sin.at[j % nbuf],
            )

        def out_dma(j):
            return pltpu.make_async_copy(
                bufs[j % nbuf],
                o_ref.at[:, pl.ds(j * chunk_cols, chunk_cols)],
                sout.at[j % nbuf],
            )

        waited_out = set()
        for j in range(min(lead, n_chunks)):
            in_dma(j).start()
        for j in range(n_chunks):
            in_dma(j).wait()
            nxt = j + lead
            if nxt < n_chunks:
                prev = nxt - nbuf
                if prev >= 0:
                    out_dma(prev).wait()
                    waited_out.add(prev)
                in_dma(nxt).start()
            out_dma(j).start()
        for j in range(n_chunks):
            if j not in waited_out:
                out_dma(j).wait()

    return body


def kernel(edge_index):
    two, n_cols = edge_index.shape
    chunk = _CHUNK_COLS if n_cols % _CHUNK_COLS == 0 else n_cols
    n_chunks = n_cols // chunk
    nbuf = min(_NBUF, n_chunks)
    lead = min(_LEAD, nbuf - 1) if nbuf > 1 else 0
    return pl.pallas_call(
        _make_body(n_chunks, chunk, nbuf, lead),
        in_specs=[pl.BlockSpec(memory_space=pltpu.MemorySpace.HBM)],
        out_specs=pl.BlockSpec(memory_space=pltpu.MemorySpace.HBM),
        scratch_shapes=(
            [pltpu.VMEM((two, chunk), edge_index.dtype) for _ in range(nbuf)]
            + [pltpu.SemaphoreType.DMA((nbuf,)), pltpu.SemaphoreType.DMA((nbuf,))]
        ),
        out_shape=jax.ShapeDtypeStruct(edge_index.shape, edge_index.dtype),
    )(edge_index)


# speedup vs baseline: 47.7683x; 1.0863x over previous
import jax
import jax.numpy as jnp
from jax.experimental import pallas as pl


def _copy_body(x_ref, o_ref):
    o_ref[...] = x_ref[...]


def _pick_chunk(n_cols):
    for chunk in (800000, 1280000, 640000, 128000, 64000, 32000, 12800, 6400, 1280, 128):
        if n_cols % chunk == 0:
            return chunk
    return None


def kernel(edge_index):
    two, n_cols = edge_index.shape
    chunk = _pick_chunk(n_cols)
    if chunk is None:
        chunk = n_cols
    grid = n_cols // chunk
    return pl.pallas_call(
        _copy_body,
        grid=(grid,),
        in_specs=[pl.BlockSpec((two, chunk), lambda i: (0, i))],
        out_specs=pl.BlockSpec((two, chunk), lambda i: (0, i)),
        out_shape=jax.ShapeDtypeStruct(edge_index.shape, edge_index.dtype),
    )(edge_index)
